# Initial kernel scaffold; baseline (speedup 1.0000x reference)
import jax
import jax.numpy as jnp
from jax.experimental import pallas as pl


def kernel(x, Wg_router, We_router, gate_w, up_w, down_w):
    raise NotImplementedError("write your pallas kernel here")


# speedup vs baseline: 1.4941x; 1.4941x over previous
import functools
import math

import jax
import jax.numpy as jnp
from jax import lax
from jax.experimental import pallas as pl
from jax.experimental.pallas import tpu as pltpu
from jax.experimental.pallas import tpu_sc as plsc

T = 2048
H = 1024
I = 2048
E = 16
G = 4
EPG = 4
TOPK = 2
CAP = int(math.ceil(T * TOPK / E * 1.25))
SLOTS = E * CAP
ZERO_ROW = SLOTS
TAB = SLOTS + 8
NEG = -1e30

TB = 256
NB = T // TB

NC = 2
NS = 16
NW = NC * NS
SPW = SLOTS // NW
TPW = T // NW
CH = 32

IB = 512
IC = I // IB


def _route_body(x_ref, wg_ref, we_ref, s1_ref, s2_ref, w1_ref, w2_ref, cnt_ref):
    i = pl.program_id(0)

    @pl.when(i == 0)
    def _():
        cnt_ref[...] = jnp.zeros_like(cnt_ref)

    x = x_ref[...]
    gl = jnp.dot(x, wg_ref[...], preferred_element_type=jnp.float32)
    gm = jnp.max(gl, axis=-1, keepdims=True)
    ge = jnp.exp(gl - gm)
    gp = ge / jnp.sum(ge, axis=-1, keepdims=True)
    gid = jnp.argmax(gp, axis=-1).astype(jnp.int32)
    gprob = jnp.max(gp, axis=-1)

    el16 = jnp.dot(x, we_ref[...], preferred_element_type=jnp.float32)
    lane = lax.broadcasted_iota(jnp.int32, (TB, E), 1)
    colmask = (lane // EPG) == gid[:, None]
    p16 = jnp.where(colmask, el16, NEG)
    m1 = jnp.max(p16, axis=-1)
    sume = jnp.sum(jnp.exp(p16 - m1[:, None]), axis=-1)
    eid1 = jnp.argmax(p16, axis=-1).astype(jnp.int32)
    p16b = jnp.where(lane == eid1[:, None], NEG, p16)
    eid2 = jnp.argmax(p16b, axis=-1).astype(jnp.int32)
    m2 = jnp.max(p16b, axis=-1)
    p1 = 1.0 / sume
    p2 = jnp.exp(m2 - m1) / sume
    den = p1 + p2
    cw1 = (p1 / den) * gprob
    cw2 = (p2 / den) * gprob

    oh1 = (lane == eid1[:, None]).astype(jnp.float32)
    oh2 = (lane == eid2[:, None]).astype(jnp.float32)
    ohs = oh1 + oh2
    r = lax.broadcasted_iota(jnp.int32, (TB, TB), 0)
    c = lax.broadcasted_iota(jnp.int32, (TB, TB), 1)
    tril = (r > c).astype(jnp.float32)
    csum = jnp.dot(tril, ohs, preferred_element_type=jnp.float32)
    tot = csum + cnt_ref[0:1, :E]
    pos1 = jnp.sum(oh1 * tot, axis=-1).astype(jnp.int32)
    pos2 = jnp.sum(oh2 * tot, axis=-1).astype(jnp.int32)
    cnt_ref[0:1, :E] = cnt_ref[0:1, :E] + jnp.sum(ohs, axis=0, keepdims=True)

    slot1 = jnp.where(pos1 < CAP, eid1 * CAP + pos1, ZERO_ROW)
    slot2 = jnp.where(pos2 < CAP, eid2 * CAP + pos2, ZERO_ROW)
    s1_ref[...] = slot1.reshape(1, 1, TB)
    s2_ref[...] = slot2.reshape(1, 1, TB)
    w1_ref[...] = cw1.reshape(1, 1, TB)
    w2_ref[...] = cw2.reshape(1, 1, TB)


def _route(x, wg, we_flat, interpret=False):
    return pl.pallas_call(
        _route_body,
        grid=(NB,),
        in_specs=[
            pl.BlockSpec((TB, H), lambda i: (i, 0)),
            pl.BlockSpec((H, G), lambda i: (0, 0)),
            pl.BlockSpec((H, E), lambda i: (0, 0)),
        ],
        out_specs=[
            pl.BlockSpec((1, 1, TB), lambda i: (i, 0, 0)),
            pl.BlockSpec((1, 1, TB), lambda i: (i, 0, 0)),
            pl.BlockSpec((1, 1, TB), lambda i: (i, 0, 0)),
            pl.BlockSpec((1, 1, TB), lambda i: (i, 0, 0)),
        ],
        out_shape=[
            jax.ShapeDtypeStruct((NB, 1, TB), jnp.int32),
            jax.ShapeDtypeStruct((NB, 1, TB), jnp.int32),
            jax.ShapeDtypeStruct((NB, 1, TB), jnp.float32),
            jax.ShapeDtypeStruct((NB, 1, TB), jnp.float32),
        ],
        scratch_shapes=[pltpu.VMEM((8, 128), jnp.float32)],
        interpret=interpret,
    )(x, wg, we_flat)


def _dispatch_body(x_hbm, s1_hbm, s2_hbm, w1_hbm, w2_hbm,
                   buf_hbm, wslot_hbm,
                   s1v, s2v, w1v, w2v, srct, wt, rows, sem):
    wid = lax.axis_index("c") * NS + lax.axis_index("s")
    base = wid * SPW
    pltpu.sync_copy(s1_hbm, s1v)
    pltpu.sync_copy(s2_hbm, s2v)
    pltpu.sync_copy(w1_hbm, w1v)
    pltpu.sync_copy(w2_hbm, w2v)

    zi = jnp.zeros((16,), jnp.int32)
    zf = jnp.zeros((16,), jnp.float32)

    def memset(k, carry):
        srct[pl.ds(k * 16, 16)] = zi
        wt[pl.ds(k * 16, 16)] = zf
        return carry
    lax.fori_loop(0, TAB // 16, memset, 0)

    def scatter(k, carry):
        t0 = k * 16
        tok = lax.iota(jnp.int32, 16) + t0
        s1 = s1v[pl.ds(t0, 16)]
        s2 = s2v[pl.ds(t0, 16)]
        plsc.store_scatter(srct, [s1], tok)
        plsc.store_scatter(srct, [s2], tok)
        plsc.store_scatter(wt, [s1], w1v[pl.ds(t0, 16)])
        plsc.store_scatter(wt, [s2], w2v[pl.ds(t0, 16)])
        return carry
    lax.fori_loop(0, T // 16, scatter, 0)

    def chunk(c, carry):
        cb = base + c * 16
        svec = srct[pl.ds(cb, 16)]
        pltpu.async_copy(x_hbm.at[svec], rows, sem).wait()
        pltpu.sync_copy(rows, buf_hbm.at[pl.ds(cb, 16)])
        return carry
    lax.fori_loop(0, SPW // 16, chunk, 0)

    pltpu.sync_copy(wt.at[pl.ds(base, SPW)], wslot_hbm.at[pl.ds(base, SPW)])


def _dispatch(x, s1, s2, w1, w2):
    mesh = plsc.VectorSubcoreMesh(core_axis_name="c", subcore_axis_name="s",
                                  num_cores=NC, num_subcores=NS)
    f = pl.kernel(
        _dispatch_body,
        out_type=[
            jax.ShapeDtypeStruct((SLOTS, H), jnp.float32),
            jax.ShapeDtypeStruct((SLOTS,), jnp.float32),
        ],
        mesh=mesh,
        scratch_types=[
            pltpu.VMEM((T,), jnp.int32),
            pltpu.VMEM((T,), jnp.int32),
            pltpu.VMEM((T,), jnp.float32),
            pltpu.VMEM((T,), jnp.float32),
            pltpu.VMEM((TAB,), jnp.int32),
            pltpu.VMEM((TAB,), jnp.float32),
            pltpu.VMEM((16, H), jnp.float32),
            pltpu.SemaphoreType.DMA,
        ],
        compiler_params=pltpu.CompilerParams(needs_layout_passes=False),
    )
    return f(x, s1, s2, w1, w2)


def _ffn_body(buf_ref, g_ref, u_ref, d_ref, ws_ref, eo_ref, acc_ref):
    e = pl.program_id(0)
    ic = pl.program_id(1)

    @pl.when(jnp.logical_and(e < E, ic == 0))
    def _():
        acc_ref[...] = jnp.zeros_like(acc_ref)

    @pl.when(e < E)
    def _():
        xb = buf_ref[...]
        g = jnp.dot(xb, g_ref[0], preferred_element_type=jnp.float32)
        u = jnp.dot(xb, u_ref[0], preferred_element_type=jnp.float32)
        h = g * lax.logistic(g) * u
        acc_ref[...] += jnp.dot(h, d_ref[0], preferred_element_type=jnp.float32)

    @pl.when(jnp.logical_and(e < E, ic == IC - 1))
    def _():
        eo_ref[...] = acc_ref[...] * ws_ref[0]

    @pl.when(jnp.logical_and(e == E, ic == IC - 1))
    def _():
        eo_ref[...] = jnp.zeros_like(eo_ref)


def _ffn(buf, gate_w, up_w, down_w, wslot3, interpret=False):
    emin = lambda e: jnp.minimum(e, E - 1)
    return pl.pallas_call(
        _ffn_body,
        grid=(E + 1, IC),
        in_specs=[
            pl.BlockSpec((CAP, H), lambda e, ic: (emin(e), 0)),
            pl.BlockSpec((1, H, IB), lambda e, ic: (emin(e), 0, ic)),
            pl.BlockSpec((1, H, IB), lambda e, ic: (emin(e), 0, ic)),
            pl.BlockSpec((1, IB, H), lambda e, ic: (emin(e), ic, 0)),
            pl.BlockSpec((1, CAP, 1), lambda e, ic: (emin(e), 0, 0)),
        ],
        out_specs=pl.BlockSpec((CAP, H), lambda e, ic: (e, 0)),
        out_shape=jax.ShapeDtypeStruct(((E + 1) * CAP, H), jnp.float32),
        scratch_shapes=[pltpu.VMEM((CAP, H), jnp.float32)],
        interpret=interpret,
    )(buf, gate_w, up_w, down_w, wslot3)


def _combine_body(eo_hbm, s1_hbm, s2_hbm, out_hbm,
                  idx1, idx2, r1, r2, sem1, sem2):
    wid = lax.axis_index("c") * NS + lax.axis_index("s")
    tb = wid * TPW

    def chunk(c, carry):
        cb = tb + c * CH
        pltpu.sync_copy(s1_hbm.at[pl.ds(cb, CH)], idx1)
        pltpu.sync_copy(s2_hbm.at[pl.ds(cb, CH)], idx2)
        cp1 = pltpu.async_copy(eo_hbm.at[idx1], r1, sem1)
        cp2 = pltpu.async_copy(eo_hbm.at[idx2], r2, sem2)
        cp1.wait()
        cp2.wait()

        def add16(q, carry2):
            j = q // (H // 16)
            off = (q % (H // 16)) * 16
            r1[j, pl.ds(off, 16)] = r1[j, pl.ds(off, 16)] + r2[j, pl.ds(off, 16)]
            return carry2
        lax.fori_loop(0, CH * (H // 16), add16, 0)
        pltpu.sync_copy(r1, out_hbm.at[pl.ds(cb, CH)])
        return carry
    lax.fori_loop(0, TPW // CH, chunk, 0)


def _combine(eo, s1, s2):
    mesh = plsc.VectorSubcoreMesh(core_axis_name="c", subcore_axis_name="s",
                                  num_cores=NC, num_subcores=NS)
    f = pl.kernel(
        _combine_body,
        out_type=jax.ShapeDtypeStruct((T, H), jnp.float32),
        mesh=mesh,
        scratch_types=[
            pltpu.VMEM((CH,), jnp.int32),
            pltpu.VMEM((CH,), jnp.int32),
            pltpu.VMEM((CH, H), jnp.float32),
            pltpu.VMEM((CH, H), jnp.float32),
            pltpu.SemaphoreType.DMA,
            pltpu.SemaphoreType.DMA,
        ],
        compiler_params=pltpu.CompilerParams(needs_layout_passes=False),
    )
    return f(eo, s1, s2)


def kernel(x, Wg_router, We_router, gate_w, up_w, down_w):
    we_flat = We_router.transpose(1, 0, 2).reshape(H, E)
    s1, s2, w1, w2 = _route(x, Wg_router, we_flat)
    s1 = s1.reshape(T)
    s2 = s2.reshape(T)
    w1 = w1.reshape(T)
    w2 = w2.reshape(T)
    buf, wslot = _dispatch(x, s1, s2, w1, w2)
    eo = _ffn(buf, gate_w, up_w, down_w, wslot.reshape(E, CAP, 1))
    out = _combine(eo, s1, s2)
    return out


# speedup vs baseline: 1.5893x; 1.0637x over previous
import functools
import math

import jax
import jax.numpy as jnp
from jax import lax
from jax.experimental import pallas as pl
from jax.experimental.pallas import tpu as pltpu
from jax.experimental.pallas import tpu_sc as plsc

T = 2048
H = 1024
I = 2048
E = 16
G = 4
EPG = 4
TOPK = 2
CAP = int(math.ceil(T * TOPK / E * 1.25))
SLOTS = E * CAP
ZERO_ROW = SLOTS
TAB = SLOTS + 8
NEG = -1e30

TB = 256
NB = T // TB

NC = 2
NS = 16
NW = NC * NS
SPW = SLOTS // NW
TPW = T // NW
CH = 32

IB = 512
IC = I // IB


def _route_body(x_ref, wg_ref, we_ref, s1_ref, s2_ref, w1_ref, w2_ref, cnt_ref):
    i = pl.program_id(0)

    @pl.when(i == 0)
    def _():
        cnt_ref[...] = jnp.zeros_like(cnt_ref)

    x = x_ref[...]
    gl = jnp.dot(x, wg_ref[...], preferred_element_type=jnp.float32)
    gm = jnp.max(gl, axis=-1, keepdims=True)
    ge = jnp.exp(gl - gm)
    gp = ge / jnp.sum(ge, axis=-1, keepdims=True)
    gid = jnp.argmax(gp, axis=-1).astype(jnp.int32)
    gprob = jnp.max(gp, axis=-1)

    el16 = jnp.dot(x, we_ref[...], preferred_element_type=jnp.float32)
    lane = lax.broadcasted_iota(jnp.int32, (TB, E), 1)
    colmask = (lane // EPG) == gid[:, None]
    p16 = jnp.where(colmask, el16, NEG)
    m1 = jnp.max(p16, axis=-1)
    sume = jnp.sum(jnp.exp(p16 - m1[:, None]), axis=-1)
    eid1 = jnp.argmax(p16, axis=-1).astype(jnp.int32)
    p16b = jnp.where(lane == eid1[:, None], NEG, p16)
    eid2 = jnp.argmax(p16b, axis=-1).astype(jnp.int32)
    m2 = jnp.max(p16b, axis=-1)
    p1 = 1.0 / sume
    p2 = jnp.exp(m2 - m1) / sume
    den = p1 + p2
    cw1 = (p1 / den) * gprob
    cw2 = (p2 / den) * gprob

    oh1 = (lane == eid1[:, None]).astype(jnp.float32)
    oh2 = (lane == eid2[:, None]).astype(jnp.float32)
    ohs = oh1 + oh2
    r = lax.broadcasted_iota(jnp.int32, (TB, TB), 0)
    c = lax.broadcasted_iota(jnp.int32, (TB, TB), 1)
    tril = (r > c).astype(jnp.float32)
    csum = jnp.dot(tril, ohs, preferred_element_type=jnp.float32)
    tot = csum + cnt_ref[0:1, :E]
    pos1 = jnp.sum(oh1 * tot, axis=-1).astype(jnp.int32)
    pos2 = jnp.sum(oh2 * tot, axis=-1).astype(jnp.int32)
    cnt_ref[0:1, :E] = cnt_ref[0:1, :E] + jnp.sum(ohs, axis=0, keepdims=True)

    slot1 = jnp.where(pos1 < CAP, eid1 * CAP + pos1, ZERO_ROW)
    slot2 = jnp.where(pos2 < CAP, eid2 * CAP + pos2, ZERO_ROW)
    s1_ref[...] = slot1.reshape(1, 1, TB)
    s2_ref[...] = slot2.reshape(1, 1, TB)
    w1_ref[...] = cw1.reshape(1, 1, TB)
    w2_ref[...] = cw2.reshape(1, 1, TB)


def _route(x, wg, we_flat, interpret=False):
    return pl.pallas_call(
        _route_body,
        grid=(NB,),
        in_specs=[
            pl.BlockSpec((TB, H), lambda i: (i, 0)),
            pl.BlockSpec((H, G), lambda i: (0, 0)),
            pl.BlockSpec((H, E), lambda i: (0, 0)),
        ],
        out_specs=[
            pl.BlockSpec((1, 1, TB), lambda i: (i, 0, 0)),
            pl.BlockSpec((1, 1, TB), lambda i: (i, 0, 0)),
            pl.BlockSpec((1, 1, TB), lambda i: (i, 0, 0)),
            pl.BlockSpec((1, 1, TB), lambda i: (i, 0, 0)),
        ],
        out_shape=[
            jax.ShapeDtypeStruct((NB, 1, TB), jnp.int32),
            jax.ShapeDtypeStruct((NB, 1, TB), jnp.int32),
            jax.ShapeDtypeStruct((NB, 1, TB), jnp.float32),
            jax.ShapeDtypeStruct((NB, 1, TB), jnp.float32),
        ],
        scratch_shapes=[pltpu.VMEM((8, 128), jnp.float32)],
        interpret=interpret,
    )(x, wg, we_flat)


def _dispatch_body(x_hbm, s1_hbm, s2_hbm, w1_hbm, w2_hbm,
                   buf_hbm, wslot_hbm,
                   s1v, s2v, w1v, w2v, srct, wt, rows, sem):
    wid = lax.axis_index("c") * NS + lax.axis_index("s")
    base = wid * SPW
    pltpu.sync_copy(s1_hbm, s1v)
    pltpu.sync_copy(s2_hbm, s2v)
    pltpu.sync_copy(w1_hbm, w1v)
    pltpu.sync_copy(w2_hbm, w2v)

    zi = jnp.zeros((16,), jnp.int32)
    zf = jnp.zeros((16,), jnp.float32)

    def memset(k, carry):
        srct[pl.ds(k * 16, 16)] = zi
        wt[pl.ds(k * 16, 16)] = zf
        return carry
    lax.fori_loop(0, TAB // 16, memset, 0)

    def scatter(k, carry):
        t0 = k * 16
        tok = lax.iota(jnp.int32, 16) + t0
        s1 = s1v[pl.ds(t0, 16)]
        s2 = s2v[pl.ds(t0, 16)]
        plsc.store_scatter(srct, [s1], tok)
        plsc.store_scatter(srct, [s2], tok)
        plsc.store_scatter(wt, [s1], w1v[pl.ds(t0, 16)])
        plsc.store_scatter(wt, [s2], w2v[pl.ds(t0, 16)])
        return carry
    lax.fori_loop(0, T // 16, scatter, 0)

    def half(h, carry):
        hb = base + h * 80
        cps = []
        for k in range(5):
            svec = srct[pl.ds(hb + k * 16, 16)]
            cps.append(pltpu.async_copy(x_hbm.at[svec],
                                        rows.at[pl.ds(k * 16, 16)], sem))
        for cp in cps:
            cp.wait()
        pltpu.sync_copy(rows, buf_hbm.at[pl.ds(hb, 80)])
        return carry
    lax.fori_loop(0, SPW // 80, half, 0)

    pltpu.sync_copy(wt.at[pl.ds(base, SPW)], wslot_hbm.at[pl.ds(base, SPW)])


def _dispatch(x, s1, s2, w1, w2):
    mesh = plsc.VectorSubcoreMesh(core_axis_name="c", subcore_axis_name="s",
                                  num_cores=NC, num_subcores=NS)
    f = pl.kernel(
        _dispatch_body,
        out_type=[
            jax.ShapeDtypeStruct((SLOTS, H), jnp.float32),
            jax.ShapeDtypeStruct((SLOTS,), jnp.float32),
        ],
        mesh=mesh,
        scratch_types=[
            pltpu.VMEM((T,), jnp.int32),
            pltpu.VMEM((T,), jnp.int32),
            pltpu.VMEM((T,), jnp.float32),
            pltpu.VMEM((T,), jnp.float32),
            pltpu.VMEM((TAB,), jnp.int32),
            pltpu.VMEM((TAB,), jnp.float32),
            pltpu.VMEM((80, H), jnp.float32),
            pltpu.SemaphoreType.DMA,
        ],
        compiler_params=pltpu.CompilerParams(needs_layout_passes=False),
    )
    return f(x, s1, s2, w1, w2)


def _ffn_body(buf_ref, g_ref, u_ref, d_ref, ws_ref, eo_ref, acc_ref):
    e = pl.program_id(0)
    ic = pl.program_id(1)

    @pl.when(jnp.logical_and(e < E, ic == 0))
    def _():
        acc_ref[...] = jnp.zeros_like(acc_ref)

    @pl.when(e < E)
    def _():
        xb = buf_ref[...]
        g = jnp.dot(xb, g_ref[0], preferred_element_type=jnp.float32)
        u = jnp.dot(xb, u_ref[0], preferred_element_type=jnp.float32)
        h = g * lax.logistic(g) * u
        acc_ref[...] += jnp.dot(h, d_ref[0], preferred_element_type=jnp.float32)

    @pl.when(jnp.logical_and(e < E, ic == IC - 1))
    def _():
        eo_ref[...] = acc_ref[...] * ws_ref[0]

    @pl.when(jnp.logical_and(e == E, ic == IC - 1))
    def _():
        eo_ref[...] = jnp.zeros_like(eo_ref)


def _ffn(buf, gate_w, up_w, down_w, wslot3, interpret=False):
    emin = lambda e: jnp.minimum(e, E - 1)
    return pl.pallas_call(
        _ffn_body,
        grid=(E + 1, IC),
        in_specs=[
            pl.BlockSpec((CAP, H), lambda e, ic: (emin(e), 0)),
            pl.BlockSpec((1, H, IB), lambda e, ic: (emin(e), 0, ic)),
            pl.BlockSpec((1, H, IB), lambda e, ic: (emin(e), 0, ic)),
            pl.BlockSpec((1, IB, H), lambda e, ic: (emin(e), ic, 0)),
            pl.BlockSpec((1, CAP, 1), lambda e, ic: (emin(e), 0, 0)),
        ],
        out_specs=pl.BlockSpec((CAP, H), lambda e, ic: (e, 0)),
        out_shape=jax.ShapeDtypeStruct(((E + 1) * CAP, H), jnp.float32),
        scratch_shapes=[pltpu.VMEM((CAP, H), jnp.float32)],
        interpret=interpret,
    )(buf, gate_w, up_w, down_w, wslot3)


def _combine_body(eo_hbm, s1_hbm, s2_hbm, out_hbm,
                  idx1, idx2, r1, r2, sem1, sem2):
    wid = lax.axis_index("c") * NS + lax.axis_index("s")
    tb = wid * TPW
    pltpu.sync_copy(s1_hbm.at[pl.ds(tb, TPW)], idx1)
    pltpu.sync_copy(s2_hbm.at[pl.ds(tb, TPW)], idx2)

    def chunk(c, carry):
        co = c * CH
        cp1 = pltpu.async_copy(eo_hbm.at[idx1.at[pl.ds(co, CH)]], r1, sem1)
        cp2 = pltpu.async_copy(eo_hbm.at[idx2.at[pl.ds(co, CH)]], r2, sem2)
        cp1.wait()
        cp2.wait()

        @plsc.parallel_loop(0, CH * (H // 16), unroll=8)
        def _(q):
            j = q // (H // 16)
            off = (q % (H // 16)) * 16
            r1[j, pl.ds(off, 16)] = r1[j, pl.ds(off, 16)] + r2[j, pl.ds(off, 16)]

        pltpu.sync_copy(r1, out_hbm.at[pl.ds(tb + co, CH)])
        return carry
    lax.fori_loop(0, TPW // CH, chunk, 0)


def _combine(eo, s1, s2):
    mesh = plsc.VectorSubcoreMesh(core_axis_name="c", subcore_axis_name="s",
                                  num_cores=NC, num_subcores=NS)
    f = pl.kernel(
        _combine_body,
        out_type=jax.ShapeDtypeStruct((T, H), jnp.float32),
        mesh=mesh,
        scratch_types=[
            pltpu.VMEM((TPW,), jnp.int32),
            pltpu.VMEM((TPW,), jnp.int32),
            pltpu.VMEM((CH, H), jnp.float32),
            pltpu.VMEM((CH, H), jnp.float32),
            pltpu.SemaphoreType.DMA,
            pltpu.SemaphoreType.DMA,
        ],
        compiler_params=pltpu.CompilerParams(needs_layout_passes=False),
    )
    return f(eo, s1, s2)


def kernel(x, Wg_router, We_router, gate_w, up_w, down_w):
    we_flat = We_router.transpose(1, 0, 2).reshape(H, E)
    s1, s2, w1, w2 = _route(x, Wg_router, we_flat)
    s1 = s1.reshape(T)
    s2 = s2.reshape(T)
    w1 = w1.reshape(T)
    w2 = w2.reshape(T)
    buf, wslot = _dispatch(x, s1, s2, w1, w2)
    eo = _ffn(buf, gate_w, up_w, down_w, wslot.reshape(E, CAP, 1))
    out = _combine(eo, s1, s2)
    return out


# speedup vs baseline: 1.7024x; 1.0712x over previous
import functools
import math

import jax
import jax.numpy as jnp
from jax import lax
from jax.experimental import pallas as pl
from jax.experimental.pallas import tpu as pltpu
from jax.experimental.pallas import tpu_sc as plsc

T = 2048
H = 1024
I = 2048
E = 16
G = 4
EPG = 4
TOPK = 2
CAP = int(math.ceil(T * TOPK / E * 1.25))
SLOTS = E * CAP
ZERO_ROW = SLOTS
TAB = SLOTS + 8
NEG = -1e30

TB = 256
NB = T // TB

NC = 2
NS = 16
NW = NC * NS
SPW = SLOTS // NW
TPW = T // NW
CH = 32

IB = 1024
IC = I // IB


def _route_body(x_ref, wg_ref, we_ref, s1_ref, s2_ref, w1_ref, w2_ref, cnt_ref):
    i = pl.program_id(0)

    @pl.when(i == 0)
    def _():
        cnt_ref[...] = jnp.zeros_like(cnt_ref)

    x = x_ref[...]
    gl = jnp.dot(x, wg_ref[...], preferred_element_type=jnp.float32)
    gm = jnp.max(gl, axis=-1, keepdims=True)
    ge = jnp.exp(gl - gm)
    gp = ge / jnp.sum(ge, axis=-1, keepdims=True)
    gid = jnp.argmax(gp, axis=-1).astype(jnp.int32)
    gprob = jnp.max(gp, axis=-1)

    el16 = jnp.dot(x, we_ref[...], preferred_element_type=jnp.float32)
    lane = lax.broadcasted_iota(jnp.int32, (TB, E), 1)
    colmask = (lane // EPG) == gid[:, None]
    p16 = jnp.where(colmask, el16, NEG)
    m1 = jnp.max(p16, axis=-1)
    sume = jnp.sum(jnp.exp(p16 - m1[:, None]), axis=-1)
    eid1 = jnp.argmax(p16, axis=-1).astype(jnp.int32)
    p16b = jnp.where(lane == eid1[:, None], NEG, p16)
    eid2 = jnp.argmax(p16b, axis=-1).astype(jnp.int32)
    m2 = jnp.max(p16b, axis=-1)
    p1 = 1.0 / sume
    p2 = jnp.exp(m2 - m1) / sume
    den = p1 + p2
    cw1 = (p1 / den) * gprob
    cw2 = (p2 / den) * gprob

    oh1 = (lane == eid1[:, None]).astype(jnp.float32)
    oh2 = (lane == eid2[:, None]).astype(jnp.float32)
    ohs = oh1 + oh2
    r = lax.broadcasted_iota(jnp.int32, (TB, TB), 0)
    c = lax.broadcasted_iota(jnp.int32, (TB, TB), 1)
    tril = (r > c).astype(jnp.float32)
    csum = jnp.dot(tril, ohs, preferred_element_type=jnp.float32)
    tot = csum + cnt_ref[0:1, :E]
    pos1 = jnp.sum(oh1 * tot, axis=-1).astype(jnp.int32)
    pos2 = jnp.sum(oh2 * tot, axis=-1).astype(jnp.int32)
    cnt_ref[0:1, :E] = cnt_ref[0:1, :E] + jnp.sum(ohs, axis=0, keepdims=True)

    slot1 = jnp.where(pos1 < CAP, eid1 * CAP + pos1, ZERO_ROW)
    slot2 = jnp.where(pos2 < CAP, eid2 * CAP + pos2, ZERO_ROW)
    s1_ref[...] = slot1.reshape(1, 1, TB)
    s2_ref[...] = slot2.reshape(1, 1, TB)
    w1_ref[...] = cw1.reshape(1, 1, TB)
    w2_ref[...] = cw2.reshape(1, 1, TB)


def _route(x, wg, we_flat, interpret=False):
    return pl.pallas_call(
        _route_body,
        grid=(NB,),
        in_specs=[
            pl.BlockSpec((TB, H), lambda i: (i, 0)),
            pl.BlockSpec((H, G), lambda i: (0, 0)),
            pl.BlockSpec((H, E), lambda i: (0, 0)),
        ],
        out_specs=[
            pl.BlockSpec((1, 1, TB), lambda i: (i, 0, 0)),
            pl.BlockSpec((1, 1, TB), lambda i: (i, 0, 0)),
            pl.BlockSpec((1, 1, TB), lambda i: (i, 0, 0)),
            pl.BlockSpec((1, 1, TB), lambda i: (i, 0, 0)),
        ],
        out_shape=[
            jax.ShapeDtypeStruct((NB, 1, TB), jnp.int32),
            jax.ShapeDtypeStruct((NB, 1, TB), jnp.int32),
            jax.ShapeDtypeStruct((NB, 1, TB), jnp.float32),
            jax.ShapeDtypeStruct((NB, 1, TB), jnp.float32),
        ],
        scratch_shapes=[pltpu.VMEM((8, 128), jnp.float32)],
        interpret=interpret,
    )(x, wg, we_flat)


def _dispatch_body(x_hbm, s1_hbm, s2_hbm, w1_hbm, w2_hbm,
                   buf_hbm, wslot_hbm,
                   s1v, s2v, w1v, w2v, srct, wt, rows0, rows1,
                   semg0, semg1, sems0, sems1):
    wid = lax.axis_index("c") * NS + lax.axis_index("s")
    base = wid * SPW
    pltpu.sync_copy(s1_hbm, s1v)
    pltpu.sync_copy(s2_hbm, s2v)
    pltpu.sync_copy(w1_hbm, w1v)
    pltpu.sync_copy(w2_hbm, w2v)

    zi = jnp.zeros((16,), jnp.int32)
    zf = jnp.zeros((16,), jnp.float32)

    @plsc.parallel_loop(0, TAB // 16, unroll=8)
    def _(k):
        srct[pl.ds(k * 16, 16)] = zi
        wt[pl.ds(k * 16, 16)] = zf

    @plsc.parallel_loop(0, T // 16, unroll=4)
    def _(k):
        t0 = k * 16
        tok = lax.iota(jnp.int32, 16) + t0
        s1 = s1v[pl.ds(t0, 16)]
        s2 = s2v[pl.ds(t0, 16)]
        plsc.store_scatter(srct, [s1], tok)
        plsc.store_scatter(srct, [s2], tok)
        plsc.store_scatter(wt, [s1], w1v[pl.ds(t0, 16)])
        plsc.store_scatter(wt, [s2], w2v[pl.ds(t0, 16)])

    rowsb = (rows0, rows1)
    semg = (semg0, semg1)
    sems = (sems0, sems1)
    gs = [None, None]
    sts = [None, None]
    for c in range(4):
        b = c % 2
        if sts[b] is not None:
            sts[b].wait()
        cb = base + c * 40
        gs[b] = pltpu.async_copy(x_hbm.at[srct.at[pl.ds(cb, 40)]],
                                 rowsb[b], semg[b])
        if c >= 1:
            pb = (c - 1) % 2
            gs[pb].wait()
            sts[pb] = pltpu.async_copy(rowsb[pb],
                                       buf_hbm.at[pl.ds(base + (c - 1) * 40, 40)],
                                       sems[pb])
    gs[1].wait()
    sts[1] = pltpu.async_copy(rowsb[1], buf_hbm.at[pl.ds(base + 120, 40)], sems[1])
    sts[0].wait()
    sts[1].wait()

    pltpu.sync_copy(wt.at[pl.ds(base, SPW)], wslot_hbm.at[pl.ds(base, SPW)])


def _dispatch(x, s1, s2, w1, w2):
    mesh = plsc.VectorSubcoreMesh(core_axis_name="c", subcore_axis_name="s",
                                  num_cores=NC, num_subcores=NS)
    f = pl.kernel(
        _dispatch_body,
        out_type=[
            jax.ShapeDtypeStruct((SLOTS, H), jnp.float32),
            jax.ShapeDtypeStruct((SLOTS,), jnp.float32),
        ],
        mesh=mesh,
        scratch_types=[
            pltpu.VMEM((T,), jnp.int32),
            pltpu.VMEM((T,), jnp.int32),
            pltpu.VMEM((T,), jnp.float32),
            pltpu.VMEM((T,), jnp.float32),
            pltpu.VMEM((TAB,), jnp.int32),
            pltpu.VMEM((TAB,), jnp.float32),
            pltpu.VMEM((40, H), jnp.float32),
            pltpu.VMEM((40, H), jnp.float32),
            pltpu.SemaphoreType.DMA,
            pltpu.SemaphoreType.DMA,
            pltpu.SemaphoreType.DMA,
            pltpu.SemaphoreType.DMA,
        ],
        compiler_params=pltpu.CompilerParams(needs_layout_passes=False),
    )
    return f(x, s1, s2, w1, w2)


def _ffn_body(buf_ref, g_ref, u_ref, d_ref, ws_ref, eo_ref, acc_ref):
    e = pl.program_id(0)
    ic = pl.program_id(1)

    @pl.when(jnp.logical_and(e < E, ic == 0))
    def _():
        acc_ref[...] = jnp.zeros_like(acc_ref)

    @pl.when(e < E)
    def _():
        xb = buf_ref[...]
        g = jnp.dot(xb, g_ref[0], preferred_element_type=jnp.float32)
        u = jnp.dot(xb, u_ref[0], preferred_element_type=jnp.float32)
        h = g * lax.logistic(g) * u
        acc_ref[...] += jnp.dot(h, d_ref[0], preferred_element_type=jnp.float32)

    @pl.when(jnp.logical_and(e < E, ic == IC - 1))
    def _():
        eo_ref[...] = acc_ref[...] * ws_ref[0]

    @pl.when(jnp.logical_and(e == E, ic == IC - 1))
    def _():
        eo_ref[...] = jnp.zeros_like(eo_ref)


def _ffn(buf, gate_w, up_w, down_w, wslot3, interpret=False):
    emin = lambda e: jnp.minimum(e, E - 1)
    return pl.pallas_call(
        _ffn_body,
        grid=(E + 1, IC),
        in_specs=[
            pl.BlockSpec((CAP, H), lambda e, ic: (emin(e), 0)),
            pl.BlockSpec((1, H, IB), lambda e, ic: (emin(e), 0, ic)),
            pl.BlockSpec((1, H, IB), lambda e, ic: (emin(e), 0, ic)),
            pl.BlockSpec((1, IB, H), lambda e, ic: (emin(e), ic, 0)),
            pl.BlockSpec((1, CAP, 1), lambda e, ic: (emin(e), 0, 0)),
        ],
        out_specs=pl.BlockSpec((CAP, H), lambda e, ic: (e, 0)),
        out_shape=jax.ShapeDtypeStruct(((E + 1) * CAP, H), jnp.float32),
        scratch_shapes=[pltpu.VMEM((CAP, H), jnp.float32)],
        interpret=interpret,
    )(buf, gate_w, up_w, down_w, wslot3)


def _combine_body(eo_hbm, s1_hbm, s2_hbm, out_hbm,
                  idx1, idx2, r1, r2, sem1, sem2):
    wid = lax.axis_index("c") * NS + lax.axis_index("s")
    tb = wid * TPW
    pltpu.sync_copy(s1_hbm.at[pl.ds(tb, TPW)], idx1)
    pltpu.sync_copy(s2_hbm.at[pl.ds(tb, TPW)], idx2)

    def chunk(c, carry):
        co = c * CH
        cp1 = pltpu.async_copy(eo_hbm.at[idx1.at[pl.ds(co, CH)]], r1, sem1)
        cp2 = pltpu.async_copy(eo_hbm.at[idx2.at[pl.ds(co, CH)]], r2, sem2)
        cp1.wait()
        cp2.wait()

        @plsc.parallel_loop(0, CH * (H // 16), unroll=8)
        def _(q):
            j = q // (H // 16)
            off = (q % (H // 16)) * 16
            r1[j, pl.ds(off, 16)] = r1[j, pl.ds(off, 16)] + r2[j, pl.ds(off, 16)]

        pltpu.sync_copy(r1, out_hbm.at[pl.ds(tb + co, CH)])
        return carry
    lax.fori_loop(0, TPW // CH, chunk, 0)


def _combine(eo, s1, s2):
    mesh = plsc.VectorSubcoreMesh(core_axis_name="c", subcore_axis_name="s",
                                  num_cores=NC, num_subcores=NS)
    f = pl.kernel(
        _combine_body,
        out_type=jax.ShapeDtypeStruct((T, H), jnp.float32),
        mesh=mesh,
        scratch_types=[
            pltpu.VMEM((TPW,), jnp.int32),
            pltpu.VMEM((TPW,), jnp.int32),
            pltpu.VMEM((CH, H), jnp.float32),
            pltpu.VMEM((CH, H), jnp.float32),
            pltpu.SemaphoreType.DMA,
            pltpu.SemaphoreType.DMA,
        ],
        compiler_params=pltpu.CompilerParams(needs_layout_passes=False),
    )
    return f(eo, s1, s2)


def kernel(x, Wg_router, We_router, gate_w, up_w, down_w):
    we_flat = We_router.transpose(1, 0, 2).reshape(H, E)
    s1, s2, w1, w2 = _route(x, Wg_router, we_flat)
    s1 = s1.reshape(T)
    s2 = s2.reshape(T)
    w1 = w1.reshape(T)
    w2 = w2.reshape(T)
    buf, wslot = _dispatch(x, s1, s2, w1, w2)
    eo = _ffn(buf, gate_w, up_w, down_w, wslot.reshape(E, CAP, 1))
    out = _combine(eo, s1, s2)
    return out


# speedup vs baseline: 2.1354x; 1.2543x over previous
import functools
import math

import jax
import jax.numpy as jnp
from jax import lax
from jax.experimental import pallas as pl
from jax.experimental.pallas import tpu as pltpu
from jax.experimental.pallas import tpu_sc as plsc

T = 2048
H = 1024
I = 2048
E = 16
G = 4
EPG = 4
TOPK = 2
CAP = int(math.ceil(T * TOPK / E * 1.25))
SLOTS = E * CAP
ZERO_ROW = SLOTS
TAB = SLOTS + 8
NEG = -1e30

TB = 256
NB = T // TB

NC = 2
NS = 16
NW = NC * NS
SPW = SLOTS // NW
TPW = T // NW
CH = 32

IB = 1024
IC = I // IB


def _route_body(x_ref, wg_ref, we_ref, s1_ref, s2_ref, w1_ref, w2_ref, cnt_ref):
    i = pl.program_id(0)

    @pl.when(i == 0)
    def _():
        cnt_ref[...] = jnp.zeros_like(cnt_ref)

    x = x_ref[...]
    gl = jnp.dot(x, wg_ref[...], preferred_element_type=jnp.float32)
    gm = jnp.max(gl, axis=-1, keepdims=True)
    ge = jnp.exp(gl - gm)
    gp = ge / jnp.sum(ge, axis=-1, keepdims=True)
    gid = jnp.argmax(gp, axis=-1).astype(jnp.int32)
    gprob = jnp.max(gp, axis=-1)

    el16 = jnp.dot(x, we_ref[...], preferred_element_type=jnp.float32)
    lane = lax.broadcasted_iota(jnp.int32, (TB, E), 1)
    colmask = (lane // EPG) == gid[:, None]
    p16 = jnp.where(colmask, el16, NEG)
    m1 = jnp.max(p16, axis=-1)
    sume = jnp.sum(jnp.exp(p16 - m1[:, None]), axis=-1)
    eid1 = jnp.argmax(p16, axis=-1).astype(jnp.int32)
    p16b = jnp.where(lane == eid1[:, None], NEG, p16)
    eid2 = jnp.argmax(p16b, axis=-1).astype(jnp.int32)
    m2 = jnp.max(p16b, axis=-1)
    p1 = 1.0 / sume
    p2 = jnp.exp(m2 - m1) / sume
    den = p1 + p2
    cw1 = (p1 / den) * gprob
    cw2 = (p2 / den) * gprob

    oh1 = (lane == eid1[:, None]).astype(jnp.float32)
    oh2 = (lane == eid2[:, None]).astype(jnp.float32)
    ohs = oh1 + oh2
    r = lax.broadcasted_iota(jnp.int32, (TB, TB), 0)
    c = lax.broadcasted_iota(jnp.int32, (TB, TB), 1)
    tril = (r > c).astype(jnp.float32)
    csum = jnp.dot(tril, ohs, preferred_element_type=jnp.float32)
    tot = csum + cnt_ref[0:1, :E]
    pos1 = jnp.sum(oh1 * tot, axis=-1).astype(jnp.int32)
    pos2 = jnp.sum(oh2 * tot, axis=-1).astype(jnp.int32)
    cnt_ref[0:1, :E] = cnt_ref[0:1, :E] + jnp.sum(ohs, axis=0, keepdims=True)

    slot1 = jnp.where(pos1 < CAP, eid1 * CAP + pos1, ZERO_ROW)
    slot2 = jnp.where(pos2 < CAP, eid2 * CAP + pos2, ZERO_ROW)
    s1_ref[...] = slot1.reshape(1, 1, TB)
    s2_ref[...] = slot2.reshape(1, 1, TB)
    w1_ref[...] = cw1.reshape(1, 1, TB)
    w2_ref[...] = cw2.reshape(1, 1, TB)


def _route(x, wg, we_flat, interpret=False):
    return pl.pallas_call(
        _route_body,
        grid=(NB,),
        in_specs=[
            pl.BlockSpec((TB, H), lambda i: (i, 0)),
            pl.BlockSpec((H, G), lambda i: (0, 0)),
            pl.BlockSpec((H, E), lambda i: (0, 0)),
        ],
        out_specs=[
            pl.BlockSpec((1, 1, TB), lambda i: (i, 0, 0)),
            pl.BlockSpec((1, 1, TB), lambda i: (i, 0, 0)),
            pl.BlockSpec((1, 1, TB), lambda i: (i, 0, 0)),
            pl.BlockSpec((1, 1, TB), lambda i: (i, 0, 0)),
        ],
        out_shape=[
            jax.ShapeDtypeStruct((NB, 1, TB), jnp.int32),
            jax.ShapeDtypeStruct((NB, 1, TB), jnp.int32),
            jax.ShapeDtypeStruct((NB, 1, TB), jnp.float32),
            jax.ShapeDtypeStruct((NB, 1, TB), jnp.float32),
        ],
        scratch_shapes=[pltpu.VMEM((8, 128), jnp.float32)],
        interpret=interpret,
    )(x, wg, we_flat)


def _dispatch_body(x_hbm, s1_hbm, s2_hbm, w1_hbm, w2_hbm,
                   buf_hbm, wslot_hbm,
                   s1v, s2v, w1v, w2v, wt, idx1, idx2, rows,
                   semr, semc1, semc2):
    wid = lax.axis_index("c") * NS + lax.axis_index("s")
    base = wid * SPW
    tb = wid * TPW
    cpr = pltpu.async_copy(x_hbm.at[pl.ds(tb, TPW)], rows, semr)
    pltpu.sync_copy(s1_hbm.at[pl.ds(tb, TPW)], idx1)
    pltpu.sync_copy(s2_hbm.at[pl.ds(tb, TPW)], idx2)
    pltpu.sync_copy(s1_hbm, s1v)
    pltpu.sync_copy(s2_hbm, s2v)
    pltpu.sync_copy(w1_hbm, w1v)
    pltpu.sync_copy(w2_hbm, w2v)

    zf = jnp.zeros((16,), jnp.float32)

    @plsc.parallel_loop(0, TAB // 16, unroll=8)
    def _(k):
        wt[pl.ds(k * 16, 16)] = zf

    @plsc.parallel_loop(0, T // 16, unroll=4)
    def _(k):
        t0 = k * 16
        s1 = s1v[pl.ds(t0, 16)]
        s2 = s2v[pl.ds(t0, 16)]
        plsc.store_scatter(wt, [s1], w1v[pl.ds(t0, 16)])
        plsc.store_scatter(wt, [s2], w2v[pl.ds(t0, 16)])

    cpr.wait()
    cp1 = pltpu.async_copy(rows, buf_hbm.at[idx1], semc1)
    cp2 = pltpu.async_copy(rows, buf_hbm.at[idx2], semc2)
    pltpu.sync_copy(wt.at[pl.ds(base, SPW)], wslot_hbm.at[pl.ds(base, SPW)])
    cp1.wait()
    cp2.wait()


def _dispatch(x, s1, s2, w1, w2):
    mesh = plsc.VectorSubcoreMesh(core_axis_name="c", subcore_axis_name="s",
                                  num_cores=NC, num_subcores=NS)
    f = pl.kernel(
        _dispatch_body,
        out_type=[
            jax.ShapeDtypeStruct(((E + 1) * CAP, H), jnp.float32),
            jax.ShapeDtypeStruct((SLOTS,), jnp.float32),
        ],
        mesh=mesh,
        scratch_types=[
            pltpu.VMEM((T,), jnp.int32),
            pltpu.VMEM((T,), jnp.int32),
            pltpu.VMEM((T,), jnp.float32),
            pltpu.VMEM((T,), jnp.float32),
            pltpu.VMEM((TAB,), jnp.float32),
            pltpu.VMEM((TPW,), jnp.int32),
            pltpu.VMEM((TPW,), jnp.int32),
            pltpu.VMEM((TPW, H), jnp.float32),
            pltpu.SemaphoreType.DMA,
            pltpu.SemaphoreType.DMA,
            pltpu.SemaphoreType.DMA,
        ],
        compiler_params=pltpu.CompilerParams(needs_layout_passes=False),
    )
    return f(x, s1, s2, w1, w2)


def _ffn_body(buf_ref, g_ref, u_ref, d_ref, ws_ref, eo_ref, acc_ref):
    e = pl.program_id(0)
    ic = pl.program_id(1)

    @pl.when(jnp.logical_and(e < E, ic == 0))
    def _():
        acc_ref[...] = jnp.zeros_like(acc_ref)

    @pl.when(e < E)
    def _():
        xb = buf_ref[...]
        g = jnp.dot(xb, g_ref[0], preferred_element_type=jnp.float32)
        u = jnp.dot(xb, u_ref[0], preferred_element_type=jnp.float32)
        h = g * lax.logistic(g) * u
        acc_ref[...] += jnp.dot(h, d_ref[0], preferred_element_type=jnp.float32)

    @pl.when(jnp.logical_and(e < E, ic == IC - 1))
    def _():
        eo_ref[...] = acc_ref[...] * ws_ref[0]

    @pl.when(jnp.logical_and(e == E, ic == IC - 1))
    def _():
        eo_ref[...] = jnp.zeros_like(eo_ref)


def _ffn(buf, gate_w, up_w, down_w, wslot3, interpret=False):
    emin = lambda e: jnp.minimum(e, E - 1)
    return pl.pallas_call(
        _ffn_body,
        grid=(E + 1, IC),
        in_specs=[
            pl.BlockSpec((CAP, H), lambda e, ic: (emin(e), 0)),
            pl.BlockSpec((1, H, IB), lambda e, ic: (emin(e), 0, ic)),
            pl.BlockSpec((1, H, IB), lambda e, ic: (emin(e), 0, ic)),
            pl.BlockSpec((1, IB, H), lambda e, ic: (emin(e), ic, 0)),
            pl.BlockSpec((1, CAP, 1), lambda e, ic: (emin(e), 0, 0)),
        ],
        out_specs=pl.BlockSpec((CAP, H), lambda e, ic: (e, 0)),
        out_shape=jax.ShapeDtypeStruct(((E + 1) * CAP, H), jnp.float32),
        scratch_shapes=[pltpu.VMEM((CAP, H), jnp.float32)],
        interpret=interpret,
    )(buf, gate_w, up_w, down_w, wslot3)


def _combine_body(eo_hbm, s1_hbm, s2_hbm, out_hbm,
                  idx1, idx2, r1, r2, sem1, sem2):
    wid = lax.axis_index("c") * NS + lax.axis_index("s")
    tb = wid * TPW
    pltpu.sync_copy(s1_hbm.at[pl.ds(tb, TPW)], idx1)
    pltpu.sync_copy(s2_hbm.at[pl.ds(tb, TPW)], idx2)

    def chunk(c, carry):
        co = c * CH
        cp1 = pltpu.async_copy(eo_hbm.at[idx1.at[pl.ds(co, CH)]], r1, sem1)
        cp2 = pltpu.async_copy(eo_hbm.at[idx2.at[pl.ds(co, CH)]], r2, sem2)
        cp1.wait()
        cp2.wait()

        @plsc.parallel_loop(0, CH * (H // 16), unroll=8)
        def _(q):
            j = q // (H // 16)
            off = (q % (H // 16)) * 16
            r1[j, pl.ds(off, 16)] = r1[j, pl.ds(off, 16)] + r2[j, pl.ds(off, 16)]

        pltpu.sync_copy(r1, out_hbm.at[pl.ds(tb + co, CH)])
        return carry
    lax.fori_loop(0, TPW // CH, chunk, 0)


def _combine(eo, s1, s2):
    mesh = plsc.VectorSubcoreMesh(core_axis_name="c", subcore_axis_name="s",
                                  num_cores=NC, num_subcores=NS)
    f = pl.kernel(
        _combine_body,
        out_type=jax.ShapeDtypeStruct((T, H), jnp.float32),
        mesh=mesh,
        scratch_types=[
            pltpu.VMEM((TPW,), jnp.int32),
            pltpu.VMEM((TPW,), jnp.int32),
            pltpu.VMEM((CH, H), jnp.float32),
            pltpu.VMEM((CH, H), jnp.float32),
            pltpu.SemaphoreType.DMA,
            pltpu.SemaphoreType.DMA,
        ],
        compiler_params=pltpu.CompilerParams(needs_layout_passes=False),
    )
    return f(eo, s1, s2)


def kernel(x, Wg_router, We_router, gate_w, up_w, down_w):
    we_flat = We_router.transpose(1, 0, 2).reshape(H, E)
    s1, s2, w1, w2 = _route(x, Wg_router, we_flat)
    s1 = s1.reshape(T)
    s2 = s2.reshape(T)
    w1 = w1.reshape(T)
    w2 = w2.reshape(T)
    buf, wslot = _dispatch(x, s1, s2, w1, w2)
    eo = _ffn(buf, gate_w, up_w, down_w, wslot.reshape(E, CAP, 1))
    out = _combine(eo, s1, s2)
    return out


# speedup vs baseline: 2.1437x; 1.0039x over previous
import functools
import math

import jax
import jax.numpy as jnp
from jax import lax
from jax.experimental import pallas as pl
from jax.experimental.pallas import tpu as pltpu
from jax.experimental.pallas import tpu_sc as plsc

T = 2048
H = 1024
I = 2048
E = 16
G = 4
EPG = 4
TOPK = 2
CAP = int(math.ceil(T * TOPK / E * 1.25))
SLOTS = E * CAP
ZERO_ROW = SLOTS
TAB = SLOTS + 8
NEG = -1e30

TB = 256
NB = T // TB

NC = 2
NS = 16
NW = NC * NS
SPW = SLOTS // NW
TPW = T // NW
CH = 32

IB = 1024
IC = I // IB


def _route_body(x_ref, wg_ref, we_ref, s1_ref, s2_ref, w1_ref, w2_ref, cnt_ref):
    i = pl.program_id(0)

    @pl.when(i == 0)
    def _():
        cnt_ref[...] = jnp.zeros_like(cnt_ref)

    x = x_ref[...]
    gl = jnp.dot(x, wg_ref[...], preferred_element_type=jnp.float32)
    gm = jnp.max(gl, axis=-1, keepdims=True)
    ge = jnp.exp(gl - gm)
    gp = ge / jnp.sum(ge, axis=-1, keepdims=True)
    gid = jnp.argmax(gp, axis=-1).astype(jnp.int32)
    gprob = jnp.max(gp, axis=-1)

    el16 = jnp.dot(x, we_ref[...], preferred_element_type=jnp.float32)
    lane = lax.broadcasted_iota(jnp.int32, (TB, E), 1)
    colmask = (lane // EPG) == gid[:, None]
    p16 = jnp.where(colmask, el16, NEG)
    m1 = jnp.max(p16, axis=-1)
    sume = jnp.sum(jnp.exp(p16 - m1[:, None]), axis=-1)
    eid1 = jnp.argmax(p16, axis=-1).astype(jnp.int32)
    p16b = jnp.where(lane == eid1[:, None], NEG, p16)
    eid2 = jnp.argmax(p16b, axis=-1).astype(jnp.int32)
    m2 = jnp.max(p16b, axis=-1)
    p1 = 1.0 / sume
    p2 = jnp.exp(m2 - m1) / sume
    den = p1 + p2
    cw1 = (p1 / den) * gprob
    cw2 = (p2 / den) * gprob

    oh1 = (lane == eid1[:, None]).astype(jnp.float32)
    oh2 = (lane == eid2[:, None]).astype(jnp.float32)
    ohs = oh1 + oh2
    r = lax.broadcasted_iota(jnp.int32, (TB, TB), 0)
    c = lax.broadcasted_iota(jnp.int32, (TB, TB), 1)
    tril = (r > c).astype(jnp.float32)
    csum = jnp.dot(tril, ohs, preferred_element_type=jnp.float32)
    tot = csum + cnt_ref[0:1, :E]
    pos1 = jnp.sum(oh1 * tot, axis=-1).astype(jnp.int32)
    pos2 = jnp.sum(oh2 * tot, axis=-1).astype(jnp.int32)
    cnt_ref[0:1, :E] = cnt_ref[0:1, :E] + jnp.sum(ohs, axis=0, keepdims=True)

    slot1 = jnp.where(pos1 < CAP, eid1 * CAP + pos1, ZERO_ROW)
    slot2 = jnp.where(pos2 < CAP, eid2 * CAP + pos2, ZERO_ROW)
    s1_ref[...] = slot1.reshape(1, 1, TB)
    s2_ref[...] = slot2.reshape(1, 1, TB)
    w1_ref[...] = cw1.reshape(1, 1, TB)
    w2_ref[...] = cw2.reshape(1, 1, TB)


def _route(x, wg, we_flat, interpret=False):
    return pl.pallas_call(
        _route_body,
        grid=(NB,),
        in_specs=[
            pl.BlockSpec((TB, H), lambda i: (i, 0)),
            pl.BlockSpec((H, G), lambda i: (0, 0)),
            pl.BlockSpec((H, E), lambda i: (0, 0)),
        ],
        out_specs=[
            pl.BlockSpec((1, 1, TB), lambda i: (i, 0, 0)),
            pl.BlockSpec((1, 1, TB), lambda i: (i, 0, 0)),
            pl.BlockSpec((1, 1, TB), lambda i: (i, 0, 0)),
            pl.BlockSpec((1, 1, TB), lambda i: (i, 0, 0)),
        ],
        out_shape=[
            jax.ShapeDtypeStruct((NB, 1, TB), jnp.int32),
            jax.ShapeDtypeStruct((NB, 1, TB), jnp.int32),
            jax.ShapeDtypeStruct((NB, 1, TB), jnp.float32),
            jax.ShapeDtypeStruct((NB, 1, TB), jnp.float32),
        ],
        scratch_shapes=[pltpu.VMEM((8, 128), jnp.float32)],
        interpret=interpret,
    )(x, wg, we_flat)


def _dispatch_body(x_hbm, s1_hbm, s2_hbm, w1_hbm, w2_hbm,
                   buf_hbm, wslot_hbm,
                   s1v, s2v, w1v, w2v, wt, idx1, idx2, rows,
                   semr, semc1, semc2):
    wid = lax.axis_index("c") * NS + lax.axis_index("s")
    base = wid * SPW
    tb = wid * TPW
    cpr = pltpu.async_copy(x_hbm.at[pl.ds(tb, TPW)], rows, semr)
    pltpu.sync_copy(s1_hbm.at[pl.ds(tb, TPW)], idx1)
    pltpu.sync_copy(s2_hbm.at[pl.ds(tb, TPW)], idx2)
    pltpu.sync_copy(s1_hbm, s1v)
    pltpu.sync_copy(s2_hbm, s2v)
    pltpu.sync_copy(w1_hbm, w1v)
    pltpu.sync_copy(w2_hbm, w2v)

    zf = jnp.zeros((16,), jnp.float32)

    @plsc.parallel_loop(0, TAB // 16, unroll=8)
    def _(k):
        wt[pl.ds(k * 16, 16)] = zf

    @plsc.parallel_loop(0, T // 16, unroll=4)
    def _(k):
        t0 = k * 16
        s1 = s1v[pl.ds(t0, 16)]
        s2 = s2v[pl.ds(t0, 16)]
        plsc.store_scatter(wt, [s1], w1v[pl.ds(t0, 16)])
        plsc.store_scatter(wt, [s2], w2v[pl.ds(t0, 16)])

    cpr.wait()
    cp1 = pltpu.async_copy(rows, buf_hbm.at[idx1], semc1)
    cp2 = pltpu.async_copy(rows, buf_hbm.at[idx2], semc2)
    pltpu.sync_copy(wt.at[pl.ds(base, SPW)], wslot_hbm.at[pl.ds(base, SPW)])
    cp1.wait()
    cp2.wait()


def _dispatch(x, s1, s2, w1, w2):
    mesh = plsc.VectorSubcoreMesh(core_axis_name="c", subcore_axis_name="s",
                                  num_cores=NC, num_subcores=NS)
    f = pl.kernel(
        _dispatch_body,
        out_type=[
            jax.ShapeDtypeStruct(((E + 1) * CAP, H), jnp.float32),
            jax.ShapeDtypeStruct((SLOTS,), jnp.float32),
        ],
        mesh=mesh,
        scratch_types=[
            pltpu.VMEM((T,), jnp.int32),
            pltpu.VMEM((T,), jnp.int32),
            pltpu.VMEM((T,), jnp.float32),
            pltpu.VMEM((T,), jnp.float32),
            pltpu.VMEM((TAB,), jnp.float32),
            pltpu.VMEM((TPW,), jnp.int32),
            pltpu.VMEM((TPW,), jnp.int32),
            pltpu.VMEM((TPW, H), jnp.float32),
            pltpu.SemaphoreType.DMA,
            pltpu.SemaphoreType.DMA,
            pltpu.SemaphoreType.DMA,
        ],
        compiler_params=pltpu.CompilerParams(needs_layout_passes=False),
    )
    return f(x, s1, s2, w1, w2)


def _ffn_body(buf_ref, g_ref, u_ref, d_ref, ws_ref, eo_ref, acc_ref):
    e = pl.program_id(0)
    ic = pl.program_id(1)

    @pl.when(jnp.logical_and(e < E, ic == 0))
    def _():
        acc_ref[...] = jnp.zeros_like(acc_ref)

    @pl.when(e < E)
    def _():
        xb = buf_ref[...].astype(jnp.bfloat16)
        g = jnp.dot(xb, g_ref[0].astype(jnp.bfloat16),
                    preferred_element_type=jnp.float32)
        u = jnp.dot(xb, u_ref[0].astype(jnp.bfloat16),
                    preferred_element_type=jnp.float32)
        h = (g * lax.logistic(g) * u).astype(jnp.bfloat16)
        acc_ref[...] += jnp.dot(h, d_ref[0].astype(jnp.bfloat16),
                                preferred_element_type=jnp.float32)

    @pl.when(jnp.logical_and(e < E, ic == IC - 1))
    def _():
        eo_ref[...] = acc_ref[...] * ws_ref[0]

    @pl.when(jnp.logical_and(e == E, ic == IC - 1))
    def _():
        eo_ref[...] = jnp.zeros_like(eo_ref)


def _ffn(buf, gate_w, up_w, down_w, wslot3, interpret=False):
    emin = lambda e: jnp.minimum(e, E - 1)
    return pl.pallas_call(
        _ffn_body,
        grid=(E + 1, IC),
        in_specs=[
            pl.BlockSpec((CAP, H), lambda e, ic: (emin(e), 0)),
            pl.BlockSpec((1, H, IB), lambda e, ic: (emin(e), 0, ic)),
            pl.BlockSpec((1, H, IB), lambda e, ic: (emin(e), 0, ic)),
            pl.BlockSpec((1, IB, H), lambda e, ic: (emin(e), ic, 0)),
            pl.BlockSpec((1, CAP, 1), lambda e, ic: (emin(e), 0, 0)),
        ],
        out_specs=pl.BlockSpec((CAP, H), lambda e, ic: (e, 0)),
        out_shape=jax.ShapeDtypeStruct(((E + 1) * CAP, H), jnp.float32),
        scratch_shapes=[pltpu.VMEM((CAP, H), jnp.float32)],
        interpret=interpret,
    )(buf, gate_w, up_w, down_w, wslot3)


def _combine_body(eo_hbm, s1_hbm, s2_hbm, out_hbm,
                  idx1, idx2, r1, r2, sem1, sem2):
    wid = lax.axis_index("c") * NS + lax.axis_index("s")
    tb = wid * TPW
    pltpu.sync_copy(s1_hbm.at[pl.ds(tb, TPW)], idx1)
    pltpu.sync_copy(s2_hbm.at[pl.ds(tb, TPW)], idx2)

    def chunk(c, carry):
        co = c * CH
        cp1 = pltpu.async_copy(eo_hbm.at[idx1.at[pl.ds(co, CH)]], r1, sem1)
        cp2 = pltpu.async_copy(eo_hbm.at[idx2.at[pl.ds(co, CH)]], r2, sem2)
        cp1.wait()
        cp2.wait()

        @plsc.parallel_loop(0, CH * (H // 16), unroll=8)
        def _(q):
            j = q // (H // 16)
            off = (q % (H // 16)) * 16
            r1[j, pl.ds(off, 16)] = r1[j, pl.ds(off, 16)] + r2[j, pl.ds(off, 16)]

        pltpu.sync_copy(r1, out_hbm.at[pl.ds(tb + co, CH)])
        return carry
    lax.fori_loop(0, TPW // CH, chunk, 0)


def _combine(eo, s1, s2):
    mesh = plsc.VectorSubcoreMesh(core_axis_name="c", subcore_axis_name="s",
                                  num_cores=NC, num_subcores=NS)
    f = pl.kernel(
        _combine_body,
        out_type=jax.ShapeDtypeStruct((T, H), jnp.float32),
        mesh=mesh,
        scratch_types=[
            pltpu.VMEM((TPW,), jnp.int32),
            pltpu.VMEM((TPW,), jnp.int32),
            pltpu.VMEM((CH, H), jnp.float32),
            pltpu.VMEM((CH, H), jnp.float32),
            pltpu.SemaphoreType.DMA,
            pltpu.SemaphoreType.DMA,
        ],
        compiler_params=pltpu.CompilerParams(needs_layout_passes=False),
    )
    return f(eo, s1, s2)


def kernel(x, Wg_router, We_router, gate_w, up_w, down_w):
    we_flat = We_router.transpose(1, 0, 2).reshape(H, E)
    s1, s2, w1, w2 = _route(x, Wg_router, we_flat)
    s1 = s1.reshape(T)
    s2 = s2.reshape(T)
    w1 = w1.reshape(T)
    w2 = w2.reshape(T)
    buf, wslot = _dispatch(x, s1, s2, w1, w2)
    eo = _ffn(buf, gate_w, up_w, down_w, wslot.reshape(E, CAP, 1))
    out = _combine(eo, s1, s2)
    return out


# speedup vs baseline: 2.1541x; 1.0048x over previous
import functools
import math

import jax
import jax.numpy as jnp
from jax import lax
from jax.experimental import pallas as pl
from jax.experimental.pallas import tpu as pltpu
from jax.experimental.pallas import tpu_sc as plsc

T = 2048
H = 1024
I = 2048
E = 16
G = 4
EPG = 4
TOPK = 2
CAP = int(math.ceil(T * TOPK / E * 1.25))
SLOTS = E * CAP
ZERO_ROW = SLOTS
TAB = SLOTS + 8
NEG = -1e30

TB = 256
NB = T // TB

NC = 2
NS = 16
NW = NC * NS
SPW = SLOTS // NW
TPW = T // NW
CH = 32

IB = 2048


def _route_body(x_ref, wg_ref, we_ref, s1_ref, s2_ref, w1_ref, w2_ref, cnt_ref):
    i = pl.program_id(0)

    @pl.when(i == 0)
    def _():
        cnt_ref[...] = jnp.zeros_like(cnt_ref)

    x = x_ref[...]
    gl = jnp.dot(x, wg_ref[...], preferred_element_type=jnp.float32)
    gm = jnp.max(gl, axis=-1, keepdims=True)
    ge = jnp.exp(gl - gm)
    gp = ge / jnp.sum(ge, axis=-1, keepdims=True)
    gid = jnp.argmax(gp, axis=-1).astype(jnp.int32)
    gprob = jnp.max(gp, axis=-1)

    el16 = jnp.concatenate(
        [jnp.dot(x, we_ref[g], preferred_element_type=jnp.float32)
         for g in range(G)], axis=1)
    lane = lax.broadcasted_iota(jnp.int32, (TB, E), 1)
    colmask = (lane // EPG) == gid[:, None]
    p16 = jnp.where(colmask, el16, NEG)
    m1 = jnp.max(p16, axis=-1)
    sume = jnp.sum(jnp.exp(p16 - m1[:, None]), axis=-1)
    eid1 = jnp.argmax(p16, axis=-1).astype(jnp.int32)
    p16b = jnp.where(lane == eid1[:, None], NEG, p16)
    eid2 = jnp.argmax(p16b, axis=-1).astype(jnp.int32)
    m2 = jnp.max(p16b, axis=-1)
    p1 = 1.0 / sume
    p2 = jnp.exp(m2 - m1) / sume
    den = p1 + p2
    cw1 = (p1 / den) * gprob
    cw2 = (p2 / den) * gprob

    oh1 = (lane == eid1[:, None]).astype(jnp.float32)
    oh2 = (lane == eid2[:, None]).astype(jnp.float32)
    ohs = oh1 + oh2
    r = lax.broadcasted_iota(jnp.int32, (TB, TB), 0)
    c = lax.broadcasted_iota(jnp.int32, (TB, TB), 1)
    tril = (r > c).astype(jnp.float32)
    csum = jnp.dot(tril, ohs, preferred_element_type=jnp.float32)
    tot = csum + cnt_ref[0:1, :E]
    pos1 = jnp.sum(oh1 * tot, axis=-1).astype(jnp.int32)
    pos2 = jnp.sum(oh2 * tot, axis=-1).astype(jnp.int32)
    cnt_ref[0:1, :E] = cnt_ref[0:1, :E] + jnp.sum(ohs, axis=0, keepdims=True)

    slot1 = jnp.where(pos1 < CAP, eid1 * CAP + pos1, ZERO_ROW)
    slot2 = jnp.where(pos2 < CAP, eid2 * CAP + pos2, ZERO_ROW)
    s1_ref[...] = slot1.reshape(1, 1, TB)
    s2_ref[...] = slot2.reshape(1, 1, TB)
    w1_ref[...] = cw1.reshape(1, 1, TB)
    w2_ref[...] = cw2.reshape(1, 1, TB)


def _route(x, wg, we_router, interpret=False):
    return pl.pallas_call(
        _route_body,
        grid=(NB,),
        in_specs=[
            pl.BlockSpec((TB, H), lambda i: (i, 0)),
            pl.BlockSpec((H, G), lambda i: (0, 0)),
            pl.BlockSpec((G, H, EPG), lambda i: (0, 0, 0)),
        ],
        out_specs=[
            pl.BlockSpec((1, 1, TB), lambda i: (i, 0, 0)),
            pl.BlockSpec((1, 1, TB), lambda i: (i, 0, 0)),
            pl.BlockSpec((1, 1, TB), lambda i: (i, 0, 0)),
            pl.BlockSpec((1, 1, TB), lambda i: (i, 0, 0)),
        ],
        out_shape=[
            jax.ShapeDtypeStruct((NB, 1, TB), jnp.int32),
            jax.ShapeDtypeStruct((NB, 1, TB), jnp.int32),
            jax.ShapeDtypeStruct((NB, 1, TB), jnp.float32),
            jax.ShapeDtypeStruct((NB, 1, TB), jnp.float32),
        ],
        scratch_shapes=[pltpu.VMEM((8, 128), jnp.float32)],
        interpret=interpret,
    )(x, wg, we_router)


def _dispatch_body(x_hbm, s1_hbm, s2_hbm, w1_hbm, w2_hbm,
                   buf_hbm, wslot_hbm,
                   s1v, s2v, w1v, w2v, wt, idx1, idx2, rows,
                   semr, semc1, semc2):
    wid = lax.axis_index("c") * NS + lax.axis_index("s")
    base = wid * SPW
    tb = wid * TPW
    cpr = pltpu.async_copy(x_hbm.at[pl.ds(tb, TPW)], rows, semr)
    pltpu.sync_copy(s1_hbm.at[pl.ds(tb, TPW)], idx1)
    pltpu.sync_copy(s2_hbm.at[pl.ds(tb, TPW)], idx2)
    pltpu.sync_copy(s1_hbm, s1v)
    pltpu.sync_copy(s2_hbm, s2v)
    pltpu.sync_copy(w1_hbm, w1v)
    pltpu.sync_copy(w2_hbm, w2v)

    zf = jnp.zeros((16,), jnp.float32)

    @plsc.parallel_loop(0, TAB // 16, unroll=8)
    def _(k):
        wt[pl.ds(k * 16, 16)] = zf

    @plsc.parallel_loop(0, T // 16, unroll=4)
    def _(k):
        t0 = k * 16
        s1 = s1v[pl.ds(t0, 16)]
        s2 = s2v[pl.ds(t0, 16)]
        plsc.store_scatter(wt, [s1], w1v[pl.ds(t0, 16)])
        plsc.store_scatter(wt, [s2], w2v[pl.ds(t0, 16)])

    cpr.wait()
    cp1 = pltpu.async_copy(rows, buf_hbm.at[idx1], semc1)
    cp2 = pltpu.async_copy(rows, buf_hbm.at[idx2], semc2)
    pltpu.sync_copy(wt.at[pl.ds(base, SPW)], wslot_hbm.at[pl.ds(base, SPW)])
    cp1.wait()
    cp2.wait()


def _dispatch(x, s1, s2, w1, w2):
    mesh = plsc.VectorSubcoreMesh(core_axis_name="c", subcore_axis_name="s",
                                  num_cores=NC, num_subcores=NS)
    f = pl.kernel(
        _dispatch_body,
        out_type=[
            jax.ShapeDtypeStruct(((E + 1) * CAP, H), jnp.float32),
            jax.ShapeDtypeStruct((SLOTS,), jnp.float32),
        ],
        mesh=mesh,
        scratch_types=[
            pltpu.VMEM((T,), jnp.int32),
            pltpu.VMEM((T,), jnp.int32),
            pltpu.VMEM((T,), jnp.float32),
            pltpu.VMEM((T,), jnp.float32),
            pltpu.VMEM((TAB,), jnp.float32),
            pltpu.VMEM((TPW,), jnp.int32),
            pltpu.VMEM((TPW,), jnp.int32),
            pltpu.VMEM((TPW, H), jnp.float32),
            pltpu.SemaphoreType.DMA,
            pltpu.SemaphoreType.DMA,
            pltpu.SemaphoreType.DMA,
        ],
        compiler_params=pltpu.CompilerParams(needs_layout_passes=False),
    )
    return f(x, s1, s2, w1, w2)


def _ffn_body(buf_ref, g_ref, u_ref, d_ref, ws_ref, eo_ref):
    e = pl.program_id(0)

    @pl.when(e < E)
    def _():
        xb = buf_ref[...].astype(jnp.bfloat16)
        g = jnp.dot(xb, g_ref[0].astype(jnp.bfloat16),
                    preferred_element_type=jnp.float32)
        u = jnp.dot(xb, u_ref[0].astype(jnp.bfloat16),
                    preferred_element_type=jnp.float32)
        h = (g * lax.logistic(g) * u).astype(jnp.bfloat16)
        eo_ref[...] = jnp.dot(h, d_ref[0].astype(jnp.bfloat16),
                              preferred_element_type=jnp.float32) * ws_ref[0]

    @pl.when(e == E)
    def _():
        eo_ref[...] = jnp.zeros_like(eo_ref)


def _ffn(buf, gate_w, up_w, down_w, wslot3, interpret=False):
    emin = lambda e: jnp.minimum(e, E - 1)
    return pl.pallas_call(
        _ffn_body,
        grid=(E + 1,),
        in_specs=[
            pl.BlockSpec((CAP, H), lambda e: (emin(e), 0)),
            pl.BlockSpec((1, H, IB), lambda e: (emin(e), 0, 0)),
            pl.BlockSpec((1, H, IB), lambda e: (emin(e), 0, 0)),
            pl.BlockSpec((1, IB, H), lambda e: (emin(e), 0, 0)),
            pl.BlockSpec((1, CAP, 1), lambda e: (emin(e), 0, 0)),
        ],
        out_specs=pl.BlockSpec((CAP, H), lambda e: (e, 0)),
        out_shape=jax.ShapeDtypeStruct(((E + 1) * CAP, H), jnp.float32),
        compiler_params=pltpu.CompilerParams(
            vmem_limit_bytes=120 * 1024 * 1024),
        interpret=interpret,
    )(buf, gate_w, up_w, down_w, wslot3)


def _combine_body(eo_hbm, s1_hbm, s2_hbm, out_hbm,
                  idx1, idx2, r1, r2, sem1, sem2):
    wid = lax.axis_index("c") * NS + lax.axis_index("s")
    tb = wid * TPW
    pltpu.sync_copy(s1_hbm.at[pl.ds(tb, TPW)], idx1)
    pltpu.sync_copy(s2_hbm.at[pl.ds(tb, TPW)], idx2)

    def chunk(c, carry):
        co = c * CH
        cp1 = pltpu.async_copy(eo_hbm.at[idx1.at[pl.ds(co, CH)]], r1, sem1)
        cp2 = pltpu.async_copy(eo_hbm.at[idx2.at[pl.ds(co, CH)]], r2, sem2)
        cp1.wait()
        cp2.wait()

        @plsc.parallel_loop(0, CH * (H // 16), unroll=8)
        def _(q):
            j = q // (H // 16)
            off = (q % (H // 16)) * 16
            r1[j, pl.ds(off, 16)] = r1[j, pl.ds(off, 16)] + r2[j, pl.ds(off, 16)]

        pltpu.sync_copy(r1, out_hbm.at[pl.ds(tb + co, CH)])
        return carry
    lax.fori_loop(0, TPW // CH, chunk, 0)


def _combine(eo, s1, s2):
    mesh = plsc.VectorSubcoreMesh(core_axis_name="c", subcore_axis_name="s",
                                  num_cores=NC, num_subcores=NS)
    f = pl.kernel(
        _combine_body,
        out_type=jax.ShapeDtypeStruct((T, H), jnp.float32),
        mesh=mesh,
        scratch_types=[
            pltpu.VMEM((TPW,), jnp.int32),
            pltpu.VMEM((TPW,), jnp.int32),
            pltpu.VMEM((CH, H), jnp.float32),
            pltpu.VMEM((CH, H), jnp.float32),
            pltpu.SemaphoreType.DMA,
            pltpu.SemaphoreType.DMA,
        ],
        compiler_params=pltpu.CompilerParams(needs_layout_passes=False),
    )
    return f(eo, s1, s2)


def kernel(x, Wg_router, We_router, gate_w, up_w, down_w):
    s1, s2, w1, w2 = _route(x, Wg_router, We_router)
    s1 = s1.reshape(T)
    s2 = s2.reshape(T)
    w1 = w1.reshape(T)
    w2 = w2.reshape(T)
    buf, wslot = _dispatch(x, s1, s2, w1, w2)
    eo = _ffn(buf, gate_w, up_w, down_w, wslot.reshape(E, CAP, 1))
    out = _combine(eo, s1, s2)
    return out
